# baseline (speedup 1.0000x reference)
import functools

import jax
import jax.numpy as jnp
from jax import lax
from jax.experimental import pallas as pl
from jax.experimental.pallas import tpu as pltpu
from jax.experimental.pallas import tpu_sc as plsc

VOCAB = 100
HIDDEN = 128

NUM_CORES = 2
NUM_SUBCORES = 16
NUM_WORKERS = NUM_CORES * NUM_SUBCORES

CH = 256
NBUF = 2
UNROLL = 2


def _emb_kernel(n_total, idx_hbm, tab_hbm, out_hbm,
                idx_v, tab_v, rows_v, *sems):
    ssem = sems
    per_w = n_total // NUM_WORKERS
    n_chunks = per_w // CH
    wid = lax.axis_index("s") * NUM_CORES + lax.axis_index("c")
    base = wid * per_w
    pltpu.sync_copy(tab_hbm, tab_v)
    pltpu.sync_copy(idx_hbm.at[pl.ds(base, per_w)], idx_v)

    def build(b, c):
        buf = rows_v.at[b]

        def one(g):
            idx16 = idx_v[pl.ds(c * CH + g * UNROLL, UNROLL)]
            for u in range(UNROLL):
                i = g * UNROLL + u
                k = idx16[u]
                for j in range(HIDDEN // 16):
                    buf[i, pl.ds(j * 16, 16)] = tab_v[k, pl.ds(j * 16, 16)]

        plsc.parallel_loop(0, CH // UNROLL, unroll=4)(one)

    def scatter(b, c):
        return pltpu.make_async_copy(
            rows_v.at[b], out_hbm.at[pl.ds(base + c * CH, CH)], ssem[b])

    for b in range(NBUF):
        build(b, b)
        scatter(b, b).start()

    def outer(i):
        for b in range(NBUF):
            c = NBUF * i + NBUF + b
            scatter(b, c - NBUF).wait()
            build(b, c)
            scatter(b, c).start()

    pl.loop(0, (n_chunks - NBUF) // NBUF)(outer)

    for b in range(NBUF):
        scatter(b, n_chunks - NBUF + b).wait()


def kernel(indices, word_embeddings):
    batch, hist = indices.shape
    n_total = batch * hist
    idx_flat = indices.reshape(n_total).astype(jnp.int32)

    mesh = plsc.VectorSubcoreMesh(
        core_axis_name="c", subcore_axis_name="s",
        num_cores=NUM_CORES, num_subcores=NUM_SUBCORES,
    )
    per_w = n_total // NUM_WORKERS

    emb = functools.partial(
        pl.kernel,
        out_type=jax.ShapeDtypeStruct((n_total, HIDDEN), jnp.float32),
        mesh=mesh,
        scratch_types=[
            pltpu.VMEM((per_w,), jnp.int32),
            pltpu.VMEM((VOCAB, HIDDEN), jnp.float32),
            pltpu.VMEM((NBUF, CH, HIDDEN), jnp.float32),
        ] + [pltpu.SemaphoreType.DMA] * NBUF,
    )(functools.partial(_emb_kernel, n_total))

    out = emb(idx_flat, word_embeddings)
    return out.reshape(batch, hist, HIDDEN)


# speedup vs baseline: 1.1512x; 1.1512x over previous
import functools

import jax
import jax.numpy as jnp
from jax import lax
from jax.experimental import pallas as pl
from jax.experimental.pallas import tpu as pltpu
from jax.experimental.pallas import tpu_sc as plsc

VOCAB = 100
HIDDEN = 128

NUM_CORES = 2
NUM_SUBCORES = 16
NUM_WORKERS = NUM_CORES * NUM_SUBCORES

CH = 256
NBUF = 2
UNROLL = 2


def _emb_kernel(n_total, idx_hbm, tab_hbm, out_hbm,
                idx_v, tab_v, rows_v, *sems):
    ssem = sems
    per_w = n_total // NUM_WORKERS
    n_chunks = per_w // CH
    wid = lax.axis_index("s") * NUM_CORES + lax.axis_index("c")
    base = wid * per_w
    pltpu.sync_copy(tab_hbm, tab_v)
    pltpu.sync_copy(idx_hbm.at[pl.ds(base, per_w)], idx_v)

    def build(b, c):
        buf = rows_v.at[b]

        def one(g):
            idx16 = idx_v[pl.ds(c * CH + g * UNROLL, UNROLL)]
            for u in range(UNROLL):
                i = g * UNROLL + u
                k = idx16[u]
                for j in range(HIDDEN // 16):
                    buf[i, pl.ds(j * 16, 16)] = tab_v[k, pl.ds(j * 16, 16)]

        plsc.parallel_loop(0, CH // UNROLL, unroll=3)(one)

    def scatter(b, c):
        return pltpu.make_async_copy(
            rows_v.at[b], out_hbm.at[pl.ds(base + c * CH, CH)], ssem[b])

    for b in range(NBUF):
        build(b, b)
        scatter(b, b).start()

    def outer(i):
        for b in range(NBUF):
            c = NBUF * i + NBUF + b
            scatter(b, c - NBUF).wait()
            build(b, c)
            scatter(b, c).start()

    pl.loop(0, (n_chunks - NBUF) // NBUF)(outer)

    for b in range(NBUF):
        scatter(b, n_chunks - NBUF + b).wait()


def kernel(indices, word_embeddings):
    batch, hist = indices.shape
    n_total = batch * hist
    idx_flat = indices.reshape(n_total).astype(jnp.int32)

    mesh = plsc.VectorSubcoreMesh(
        core_axis_name="c", subcore_axis_name="s",
        num_cores=NUM_CORES, num_subcores=NUM_SUBCORES,
    )
    per_w = n_total // NUM_WORKERS

    emb = functools.partial(
        pl.kernel,
        out_type=jax.ShapeDtypeStruct((n_total, HIDDEN), jnp.float32),
        mesh=mesh,
        scratch_types=[
            pltpu.VMEM((per_w,), jnp.int32),
            pltpu.VMEM((VOCAB, HIDDEN), jnp.float32),
            pltpu.VMEM((NBUF, CH, HIDDEN), jnp.float32),
        ] + [pltpu.SemaphoreType.DMA] * NBUF,
    )(functools.partial(_emb_kernel, n_total))

    out = emb(idx_flat, word_embeddings)
    return out.reshape(batch, hist, HIDDEN)


# speedup vs baseline: 1.2824x; 1.1139x over previous
import functools

import jax
import jax.numpy as jnp
from jax import lax
from jax.experimental import pallas as pl
from jax.experimental.pallas import tpu as pltpu
from jax.experimental.pallas import tpu_sc as plsc

VOCAB = 100
HIDDEN = 128

NUM_CORES = 2
NUM_SUBCORES = 16
NUM_WORKERS = NUM_CORES * NUM_SUBCORES

CH = 256
NBUF = 2
UNROLL = 2


def _emb_kernel(n_total, idx_hbm, tab_hbm, out_hbm,
                idx_v, tab_v, rows_v, *sems):
    ssem = sems
    per_w = n_total // NUM_WORKERS
    n_chunks = per_w // CH
    wid = lax.axis_index("s") * NUM_CORES + lax.axis_index("c")
    base = wid * per_w
    pltpu.sync_copy(tab_hbm, tab_v)
    pltpu.sync_copy(idx_hbm.at[pl.ds(base, per_w)], idx_v)

    def build(b, c):
        buf = rows_v.at[b]

        def one(g):
            idx16 = idx_v[pl.ds(c * CH + g * UNROLL, UNROLL)]
            for u in range(UNROLL):
                i = g * UNROLL + u
                k = idx16[u]
                for j in range(HIDDEN // 16):
                    buf[i, pl.ds(j * 16, 16)] = tab_v[k, pl.ds(j * 16, 16)]

        plsc.parallel_loop(0, CH // UNROLL, unroll=2)(one)

    def scatter(b, c):
        return pltpu.make_async_copy(
            rows_v.at[b], out_hbm.at[pl.ds(base + c * CH, CH)], ssem[b])

    for b in range(NBUF):
        build(b, b)
        scatter(b, b).start()

    def outer(i):
        for b in range(NBUF):
            c = NBUF * i + NBUF + b
            scatter(b, c - NBUF).wait()
            build(b, c)
            scatter(b, c).start()

    pl.loop(0, (n_chunks - NBUF) // NBUF)(outer)

    for b in range(NBUF):
        scatter(b, n_chunks - NBUF + b).wait()


def kernel(indices, word_embeddings):
    batch, hist = indices.shape
    n_total = batch * hist
    idx_flat = indices.reshape(n_total).astype(jnp.int32)

    mesh = plsc.VectorSubcoreMesh(
        core_axis_name="c", subcore_axis_name="s",
        num_cores=NUM_CORES, num_subcores=NUM_SUBCORES,
    )
    per_w = n_total // NUM_WORKERS

    emb = functools.partial(
        pl.kernel,
        out_type=jax.ShapeDtypeStruct((n_total, HIDDEN), jnp.float32),
        mesh=mesh,
        scratch_types=[
            pltpu.VMEM((per_w,), jnp.int32),
            pltpu.VMEM((VOCAB, HIDDEN), jnp.float32),
            pltpu.VMEM((NBUF, CH, HIDDEN), jnp.float32),
        ] + [pltpu.SemaphoreType.DMA] * NBUF,
    )(functools.partial(_emb_kernel, n_total))

    out = emb(idx_flat, word_embeddings)
    return out.reshape(batch, hist, HIDDEN)
